# Initial kernel scaffold; baseline (speedup 1.0000x reference)
import functools

import jax
import jax.numpy as jnp
from jax import lax
from jax.experimental import pallas as pl
from jax.experimental.pallas import tpu as pltpu
from jax.experimental.pallas import tpu_sc as plsc

NC = 2
NS = 16
NW = NC * NS
CHUNK = 128


def _make_embed(N, D):
    BPW = N // NW
    NCHUNK = BPW // CHUNK
    mesh = plsc.VectorSubcoreMesh(core_axis_name="c", subcore_axis_name="s")

    @functools.partial(
        pl.kernel,
        out_type=jax.ShapeDtypeStruct((N, D), jnp.float32),
        mesh=mesh,
        scratch_types=[
            pltpu.VMEM((NCHUNK, CHUNK), jnp.int32),
            pltpu.VMEM((2, CHUNK, D), jnp.float32),
            pltpu.SemaphoreType.DMA,
        ],
    )
    def embed(z_hbm, w_hbm, out_hbm, idx_v, rows_v, gsem):
        wid = lax.axis_index("s") * NC + lax.axis_index("c")
        base_chunk = wid * NCHUNK

        pltpu.sync_copy(z_hbm.at[pl.ds(base_chunk, NCHUNK)], idx_v)

        def sub1(c, carry):
            for k in range(CHUNK // 16):
                sl = pl.ds(k * 16, 16)
                idx_v[c, sl] = idx_v[c, sl] - 1
            return carry

        lax.fori_loop(0, NCHUNK, sub1, 0)

        def chunk_body(c, carry):
            pltpu.async_copy(w_hbm.at[idx_v.at[c]], rows_v.at[0], gsem).wait()
            pltpu.sync_copy(
                rows_v.at[0], out_hbm.at[pl.ds((base_chunk + c) * CHUNK, CHUNK)]
            )
            return carry

        lax.fori_loop(0, NCHUNK, chunk_body, 0)

    return embed


def kernel(z, weight):
    (N,) = z.shape
    _, D = weight.shape
    z2 = z.reshape(N // CHUNK, CHUNK)
    return _make_embed(N, D)(z2, weight)


# speedup vs baseline: 3.5896x; 3.5896x over previous
import functools

import jax
import jax.numpy as jnp
from jax import lax
from jax.experimental import pallas as pl
from jax.experimental.pallas import tpu as pltpu
from jax.experimental.pallas import tpu_sc as plsc

NC = 2
NS = 16
NW = NC * NS
CHUNK = 128


def _make_embed(N, D):
    BPW = N // NW
    NCHUNK = BPW // CHUNK
    mesh = plsc.VectorSubcoreMesh(
        core_axis_name="c", subcore_axis_name="s", num_cores=NC, num_subcores=NS
    )

    @functools.partial(
        pl.kernel,
        out_type=jax.ShapeDtypeStruct((N, D), jnp.float32),
        mesh=mesh,
        scratch_types=[
            pltpu.VMEM((NCHUNK, CHUNK), jnp.int32),
            pltpu.VMEM((2, CHUNK, D), jnp.float32),
            pltpu.SemaphoreType.DMA,
        ],
    )
    def embed(z_hbm, w_hbm, out_hbm, idx_v, rows_v, gsem):
        wid = lax.axis_index("s") * NC + lax.axis_index("c")
        base_chunk = wid * NCHUNK

        pltpu.sync_copy(z_hbm.at[pl.ds(base_chunk, NCHUNK)], idx_v)

        def sub1(c, carry):
            for k in range(CHUNK // 16):
                sl = pl.ds(k * 16, 16)
                idx_v[c, sl] = idx_v[c, sl] - 1
            return carry

        lax.fori_loop(0, NCHUNK, sub1, 0)

        def chunk_body(c, carry):
            pltpu.async_copy(w_hbm.at[idx_v.at[c]], rows_v.at[0], gsem).wait()
            pltpu.sync_copy(
                rows_v.at[0], out_hbm.at[pl.ds((base_chunk + c) * CHUNK, CHUNK)]
            )
            return carry

        lax.fori_loop(0, NCHUNK, chunk_body, 0)

    return embed


def kernel(z, weight):
    (N,) = z.shape
    _, D = weight.shape
    z2 = z.reshape(N // CHUNK, CHUNK)
    return _make_embed(N, D)(z2, weight)


# speedup vs baseline: 3.7161x; 1.0352x over previous
import functools

import jax
import jax.numpy as jnp
from jax import lax
from jax.experimental import pallas as pl
from jax.experimental.pallas import tpu as pltpu
from jax.experimental.pallas import tpu_sc as plsc

NC = 2
NS = 16
NW = NC * NS
CHUNK = 128
NBUF = 4


def _make_embed(N, D):
    BPW = N // NW
    NCHUNK = BPW // CHUNK
    NGRP = NCHUNK // NBUF
    mesh = plsc.VectorSubcoreMesh(
        core_axis_name="c", subcore_axis_name="s", num_cores=NC, num_subcores=NS
    )

    @functools.partial(
        pl.kernel,
        out_type=jax.ShapeDtypeStruct((N, D), jnp.float32),
        mesh=mesh,
        scratch_types=[
            pltpu.VMEM((NCHUNK, CHUNK), jnp.int32),
            pltpu.VMEM((NBUF, CHUNK, D), jnp.float32),
        ]
        + [pltpu.SemaphoreType.DMA] * (2 * NBUF),
    )
    def embed(z_hbm, w_hbm, out_hbm, idx_v, rows_v, *sems):
        gsems, wsems = sems[:NBUF], sems[NBUF:]
        wid = lax.axis_index("s") * NC + lax.axis_index("c")
        base_chunk = wid * NCHUNK

        pltpu.sync_copy(z_hbm.at[pl.ds(base_chunk, NCHUNK)], idx_v)

        def sub1(c):
            for k in range(CHUNK // 16):
                sl = pl.ds(k * 16, 16)
                idx_v[c, sl] = idx_v[c, sl] - 1

        def gather(c, b):
            return pltpu.make_async_copy(
                w_hbm.at[idx_v.at[c]], rows_v.at[b], gsems[b]
            )

        def write(c, b):
            return pltpu.make_async_copy(
                rows_v.at[b],
                out_hbm.at[pl.ds((base_chunk + c) * CHUNK, CHUNK)],
                wsems[b],
            )

        for b in range(NBUF):
            sub1(b)
            gather(b, b).start()

        def group(g, carry):
            base = g * NBUF
            for b in range(NBUF):
                gather(base + b, b).wait()
                write(base + b, b).start()
            for b in range(NBUF):
                c = base + b
                write(c, b).wait()

                @pl.when(c + NBUF < NCHUNK)
                def _():
                    sub1(c + NBUF)
                    gather(c + NBUF, b).start()

            return carry

        lax.fori_loop(0, NGRP, group, 0)

    return embed


def kernel(z, weight):
    (N,) = z.shape
    _, D = weight.shape
    z2 = z.reshape(N // CHUNK, CHUNK)
    return _make_embed(N, D)(z2, weight)


# speedup vs baseline: 4.5617x; 1.2275x over previous
import functools

import jax
import jax.numpy as jnp
from jax import lax
from jax.experimental import pallas as pl
from jax.experimental.pallas import tpu as pltpu
from jax.experimental.pallas import tpu_sc as plsc

NC = 2
NS = 16
NW = NC * NS
CHUNK = 256
NBUF = 2


def _make_embed(N, V, D):
    BPW = N // NW
    NCHUNK = BPW // CHUNK
    NGRP = NCHUNK // NBUF
    mesh = plsc.VectorSubcoreMesh(
        core_axis_name="c", subcore_axis_name="s", num_cores=NC, num_subcores=NS
    )

    @functools.partial(
        pl.kernel,
        out_type=jax.ShapeDtypeStruct((N, D), jnp.float32),
        mesh=mesh,
        scratch_types=[
            pltpu.VMEM((V, D), jnp.float32),
            pltpu.VMEM((NCHUNK, CHUNK), jnp.int32),
            pltpu.VMEM((NBUF, CHUNK, D), jnp.float32),
        ]
        + [pltpu.SemaphoreType.DMA] * NBUF,
    )
    def embed(z_hbm, w_hbm, out_hbm, table_v, idx_v, rows_v, *wsems):
        wid = lax.axis_index("s") * NC + lax.axis_index("c")
        base_chunk = wid * NCHUNK

        pltpu.sync_copy(w_hbm, table_v)
        pltpu.sync_copy(z_hbm.at[pl.ds(base_chunk, NCHUNK)], idx_v)

        def write(c, b):
            return pltpu.make_async_copy(
                rows_v.at[b],
                out_hbm.at[pl.ds((base_chunk + c) * CHUNK, CHUNK)],
                wsems[b],
            )

        def compute_chunk(c, b):
            def row16(q, carry):
                zv = idx_v[c, pl.ds(q * 16, 16)] - 1
                for u in range(16):
                    zr = zv[u]
                    for k in range(D // 16):
                        sl = pl.ds(k * 16, 16)
                        rows_v[b, q * 16 + u, sl] = table_v[zr, sl]
                return carry

            lax.fori_loop(0, CHUNK // 16, row16, 0)

        def group(g, carry):
            base = g * NBUF
            for b in range(NBUF):
                c = base + b

                @pl.when(g > 0)
                def _():
                    write(c - NBUF, b).wait()

                compute_chunk(c, b)
                write(c, b).start()
            return carry

        lax.fori_loop(0, NGRP, group, 0)
        for b in range(NBUF):
            write(NCHUNK - NBUF + b, b).wait()

    return embed


def kernel(z, weight):
    (N,) = z.shape
    V, D = weight.shape
    z2 = z.reshape(N // CHUNK, CHUNK)
    return _make_embed(N, V, D)(z2, weight)


# speedup vs baseline: 12.1966x; 2.6737x over previous
import functools

import jax
import jax.numpy as jnp
from jax import lax
from jax.experimental import pallas as pl
from jax.experimental.pallas import tpu as pltpu
from jax.experimental.pallas import tpu_sc as plsc

NC = 2
NS = 16
NW = NC * NS
CHUNK = 256
NBUF = 2


def _make_embed(N, V, D):
    BPW = N // NW
    NCHUNK = BPW // CHUNK
    NGRP = NCHUNK // NBUF
    mesh = plsc.VectorSubcoreMesh(
        core_axis_name="c", subcore_axis_name="s", num_cores=NC, num_subcores=NS
    )

    @functools.partial(
        pl.kernel,
        out_type=jax.ShapeDtypeStruct((N, D), jnp.float32),
        mesh=mesh,
        scratch_types=[
            pltpu.VMEM((V, D), jnp.float32),
            pltpu.VMEM((NCHUNK, CHUNK), jnp.int32),
            pltpu.VMEM((NBUF, CHUNK, D), jnp.float32),
        ]
        + [pltpu.SemaphoreType.DMA] * NBUF,
    )
    def embed(z_hbm, w_hbm, out_hbm, table_v, idx_v, rows_v, *wsems):
        wid = lax.axis_index("s") * NC + lax.axis_index("c")
        base_chunk = wid * NCHUNK

        pltpu.sync_copy(w_hbm, table_v)
        pltpu.sync_copy(z_hbm.at[pl.ds(base_chunk, NCHUNK)], idx_v)

        def write(c, b):
            return pltpu.make_async_copy(
                rows_v.at[b],
                out_hbm.at[pl.ds((base_chunk + c) * CHUNK, CHUNK)],
                wsems[b],
            )

        def compute_chunk(c, b):
            @plsc.parallel_loop(0, CHUNK // 16, 1)
            def row16(q):
                zv = idx_v[c, pl.ds(q * 16, 16)] - 1
                for u in range(16):
                    zr = zv[u]
                    for k in range(D // 16):
                        sl = pl.ds(k * 16, 16)
                        rows_v[b, q * 16 + u, sl] = table_v[zr, sl]

        def group(g, carry):
            base = g * NBUF
            for b in range(NBUF):
                c = base + b

                @pl.when(g > 0)
                def _():
                    write(c - NBUF, b).wait()

                compute_chunk(c, b)
                write(c, b).start()
            return carry

        lax.fori_loop(0, NGRP, group, 0)
        for b in range(NBUF):
            write(NCHUNK - NBUF + b, b).wait()

    return embed


def kernel(z, weight):
    (N,) = z.shape
    V, D = weight.shape
    z2 = z.reshape(N // CHUNK, CHUNK)
    return _make_embed(N, V, D)(z2, weight)
